# baseline (speedup 1.0000x reference)
import functools

import jax
import jax.numpy as jnp
import numpy as np
from jax import lax
from jax.experimental import pallas as pl
from jax.experimental.pallas import tpu as pltpu
from jax.experimental.pallas import tpu_sc as plsc

_NC = 2
_NS = 16
_NW = _NC * _NS
_SUB = 32768


def _line_points(x0, y0, x1, y1, H, W):
    pts = []
    dx, dy = abs(x1 - x0), abs(y1 - y0)
    sx = 1 if x0 < x1 else -1
    sy = 1 if y0 < y1 else -1
    err = dx - dy
    cx, cy = x0, y0
    for _ in range(max(dx, dy) + 1):
        if 0 <= cy < H and 0 <= cx < W:
            pts.append((cy, cx))
        e2 = 2 * err
        if e2 > -dy:
            err -= dy
            cx += sx
        if e2 < dx:
            err += dx
            cy += sy
    return pts


@functools.lru_cache(maxsize=None)
def _line_mask(B, H, W):
    rng = np.random.default_rng(0)
    mask = np.zeros((B, 1, H, W), dtype=np.bool_)
    for b in range(B):
        for _ in range(6):
            y0 = int(rng.integers(0, H))
            x0 = int(rng.integers(0, W))
            y1 = int(rng.integers(0, H))
            x1 = int(rng.integers(0, W))
            for (cy, cx) in _line_points(x0, y0, x1, y1, H, W):
                mask[b, 0, cy, cx] = True
    return mask


@functools.lru_cache(maxsize=None)
def _scatter_indices(B, C, H, W):
    mask = _line_mask(B, H, W)[:, 0]
    bs, ys, xs = np.nonzero(mask)
    flat = (bs[:, None] * C + np.arange(C)[None, :]) * (H * W) + (
        ys * W + xs
    )[:, None]
    flat = np.sort(flat.ravel()).astype(np.int64)
    chunk = (B * C * H * W) // _NW
    per_tile = [flat[(flat >= w * chunk) & (flat < (w + 1) * chunk)]
                for w in range(_NW)]
    nrows = max(1, -(-max(len(p) for p in per_tile) // 128))
    out = np.empty((_NW, nrows * 128), dtype=np.int32)
    for w, p in enumerate(per_tile):
        pad_val = p[-1] if len(p) else w * chunk
        out[w, :len(p)] = p
        out[w, len(p):] = pad_val
    return out


def _sc_body(nsub, chunk, x_hbm, idx_hbm, val_hbm, out_hbm,
             buf0, buf1, idx_v, val_v,
             sem_in0, sem_in1, sem_out0, sem_out1, sem_sc):
    cid = lax.axis_index("c")
    sid = lax.axis_index("s")
    wid = sid * _NC + cid
    base = wid * chunk

    pltpu.sync_copy(idx_hbm.at[wid], idx_v)
    pltpu.sync_copy(val_hbm, val_v)

    def cp_in(g, buf, sem):
        return pltpu.make_async_copy(
            x_hbm.at[pl.ds(base + g * _SUB, _SUB)], buf, sem)

    def cp_out(g, buf, sem):
        return pltpu.make_async_copy(
            buf, out_hbm.at[pl.ds(base + g * _SUB, _SUB)], sem)

    def clip_buf(buf):
        @pl.loop(0, _SUB // 16, unroll=8)
        def _(i):
            v = buf[pl.ds(i * 16, 16)]
            buf[pl.ds(i * 16, 16)] = jnp.minimum(jnp.maximum(v, 0.0), 1.0)

    cp_in(0, buf0, sem_in0).start()

    @pl.loop(0, nsub, step=2)
    def _(g):
        cp_in(g, buf0, sem_in0).wait()

        @pl.when(g + 1 < nsub)
        def _():
            @pl.when(g > 0)
            def _():
                cp_out(g - 1, buf1, sem_out1).wait()
            cp_in(g + 1, buf1, sem_in1).start()

        clip_buf(buf0)
        cp_out(g, buf0, sem_out0).start()

        @pl.when(g + 1 < nsub)
        def _():
            cp_in(g + 1, buf1, sem_in1).wait()

            @pl.when(g + 2 < nsub)
            def _():
                cp_out(g, buf0, sem_out0).wait()
                cp_in(g + 2, buf0, sem_in0).start()

            clip_buf(buf1)
            cp_out(g + 1, buf1, sem_out1).start()

    cp_out(nsub - 2, buf0, sem_out0).wait()
    cp_out(nsub - 1, buf1, sem_out1).wait()

    pltpu.async_copy(val_v, out_hbm.at[idx_v], sem_sc).wait()


def _kernel_sc(x):
    B, C, H, W = x.shape
    n = B * C * H * W
    chunk = n // _NW
    assert chunk % _SUB == 0
    nsub = chunk // _SUB
    idx = jnp.asarray(_scatter_indices(B, C, H, W))
    nidx = idx.shape[1]
    vals = jnp.full((nidx,), 0.05, dtype=jnp.float32)
    mesh = plsc.VectorSubcoreMesh(
        core_axis_name="c", subcore_axis_name="s",
        num_cores=_NC, num_subcores=_NS)
    run = pl.kernel(
        functools.partial(_sc_body, nsub, chunk),
        out_type=jax.ShapeDtypeStruct((n,), jnp.float32),
        mesh=mesh,
        scratch_types=[
            pltpu.VMEM((_SUB,), jnp.float32),
            pltpu.VMEM((_SUB,), jnp.float32),
            pltpu.VMEM((nidx,), jnp.int32),
            pltpu.VMEM((nidx,), jnp.float32),
            pltpu.SemaphoreType.DMA,
            pltpu.SemaphoreType.DMA,
            pltpu.SemaphoreType.DMA,
            pltpu.SemaphoreType.DMA,
            pltpu.SemaphoreType.DMA,
        ],
    )
    out = run(x.reshape(n), idx, vals)
    return out.reshape(B, C, H, W)


def _fused_kernel(x_ref, m_ref, o_ref):
    o_ref[...] = jnp.where(
        m_ref[...], jnp.float32(0.05), jnp.clip(x_ref[...], 0.0, 1.0)
    )


def _kernel_tc(x):
    B, C, H, W = x.shape
    mask = jnp.asarray(_line_mask(B, H, W))
    HB = 64
    grid = (B, H // HB)
    return pl.pallas_call(
        _fused_kernel,
        grid=grid,
        in_specs=[
            pl.BlockSpec((1, C, HB, W), lambda b, h: (b, 0, h, 0)),
            pl.BlockSpec((1, 1, HB, W), lambda b, h: (b, 0, h, 0)),
        ],
        out_specs=pl.BlockSpec((1, C, HB, W), lambda b, h: (b, 0, h, 0)),
        out_shape=jax.ShapeDtypeStruct((B, C, H, W), x.dtype),
    )(x, mask)


def kernel(x):
    return _kernel_sc(x)


# speedup vs baseline: 1.0079x; 1.0079x over previous
import functools

import jax
import jax.numpy as jnp
import numpy as np
from jax import lax
from jax.experimental import pallas as pl
from jax.experimental.pallas import tpu as pltpu
from jax.experimental.pallas import tpu_sc as plsc

_NC = 2
_NS = 16
_NW = _NC * _NS
_SUB = 32768


def _line_points(x0, y0, x1, y1, H, W):
    pts = []
    dx, dy = abs(x1 - x0), abs(y1 - y0)
    sx = 1 if x0 < x1 else -1
    sy = 1 if y0 < y1 else -1
    err = dx - dy
    cx, cy = x0, y0
    for _ in range(max(dx, dy) + 1):
        if 0 <= cy < H and 0 <= cx < W:
            pts.append((cy, cx))
        e2 = 2 * err
        if e2 > -dy:
            err -= dy
            cx += sx
        if e2 < dx:
            err += dx
            cy += sy
    return pts


@functools.lru_cache(maxsize=None)
def _line_mask(B, H, W):
    rng = np.random.default_rng(0)
    mask = np.zeros((B, 1, H, W), dtype=np.bool_)
    for b in range(B):
        for _ in range(6):
            y0 = int(rng.integers(0, H))
            x0 = int(rng.integers(0, W))
            y1 = int(rng.integers(0, H))
            x1 = int(rng.integers(0, W))
            for (cy, cx) in _line_points(x0, y0, x1, y1, H, W):
                mask[b, 0, cy, cx] = True
    return mask


@functools.lru_cache(maxsize=None)
def _scatter_indices(B, C, H, W):
    mask = _line_mask(B, H, W)[:, 0]
    bs, ys, xs = np.nonzero(mask)
    flat = (bs[:, None] * C + np.arange(C)[None, :]) * (H * W) + (
        ys * W + xs
    )[:, None]
    flat = np.sort(flat.ravel()).astype(np.int64)
    chunk = (B * C * H * W) // _NW
    per_tile = [flat[(flat >= w * chunk) & (flat < (w + 1) * chunk)]
                for w in range(_NW)]
    nrows = max(1, -(-max(len(p) for p in per_tile) // 128))
    out = np.empty((_NW, nrows * 128), dtype=np.int32)
    for w, p in enumerate(per_tile):
        pad_val = p[-1] if len(p) else w * chunk
        out[w, :len(p)] = p
        out[w, len(p):] = pad_val
    return out


def _sc_body(nsub, chunk, x_hbm, idx_hbm, val_hbm, out_hbm,
             buf0, buf1, idx_v, val_v,
             sem_in0, sem_in1, sem_out0, sem_out1, sem_sc):
    cid = lax.axis_index("c")
    sid = lax.axis_index("s")
    wid = sid * _NC + cid
    base = wid * chunk

    pltpu.sync_copy(idx_hbm.at[wid], idx_v)
    pltpu.sync_copy(val_hbm, val_v)

    def cp_in(g, buf, sem):
        return pltpu.make_async_copy(
            x_hbm.at[pl.ds(base + g * _SUB, _SUB)], buf, sem)

    def cp_out(g, buf, sem):
        return pltpu.make_async_copy(
            buf, out_hbm.at[pl.ds(base + g * _SUB, _SUB)], sem)

    def clip_buf(buf):
        @plsc.parallel_loop(0, _SUB // 16, unroll=8)
        def _(i):
            v = buf[pl.ds(i * 16, 16)]
            buf[pl.ds(i * 16, 16)] = jnp.minimum(jnp.maximum(v, 0.0), 1.0)

    cp_in(0, buf0, sem_in0).start()

    @pl.loop(0, nsub, step=2)
    def _(g):
        cp_in(g, buf0, sem_in0).wait()

        @pl.when(g + 1 < nsub)
        def _():
            @pl.when(g > 0)
            def _():
                cp_out(g - 1, buf1, sem_out1).wait()
            cp_in(g + 1, buf1, sem_in1).start()

        clip_buf(buf0)
        cp_out(g, buf0, sem_out0).start()

        @pl.when(g + 1 < nsub)
        def _():
            cp_in(g + 1, buf1, sem_in1).wait()

            @pl.when(g + 2 < nsub)
            def _():
                cp_out(g, buf0, sem_out0).wait()
                cp_in(g + 2, buf0, sem_in0).start()

            clip_buf(buf1)
            cp_out(g + 1, buf1, sem_out1).start()

    cp_out(nsub - 2, buf0, sem_out0).wait()
    cp_out(nsub - 1, buf1, sem_out1).wait()

    pltpu.async_copy(val_v, out_hbm.at[idx_v], sem_sc).wait()


def _kernel_sc(x):
    B, C, H, W = x.shape
    n = B * C * H * W
    chunk = n // _NW
    assert chunk % _SUB == 0
    nsub = chunk // _SUB
    idx = jnp.asarray(_scatter_indices(B, C, H, W))
    nidx = idx.shape[1]
    vals = jnp.full((nidx,), 0.05, dtype=jnp.float32)
    mesh = plsc.VectorSubcoreMesh(
        core_axis_name="c", subcore_axis_name="s",
        num_cores=_NC, num_subcores=_NS)
    run = pl.kernel(
        functools.partial(_sc_body, nsub, chunk),
        out_type=jax.ShapeDtypeStruct((n,), jnp.float32),
        mesh=mesh,
        scratch_types=[
            pltpu.VMEM((_SUB,), jnp.float32),
            pltpu.VMEM((_SUB,), jnp.float32),
            pltpu.VMEM((nidx,), jnp.int32),
            pltpu.VMEM((nidx,), jnp.float32),
            pltpu.SemaphoreType.DMA,
            pltpu.SemaphoreType.DMA,
            pltpu.SemaphoreType.DMA,
            pltpu.SemaphoreType.DMA,
            pltpu.SemaphoreType.DMA,
        ],
    )
    out = run(x.reshape(n), idx, vals)
    return out.reshape(B, C, H, W)


def _fused_kernel(x_ref, m_ref, o_ref):
    o_ref[...] = jnp.where(
        m_ref[...], jnp.float32(0.05), jnp.clip(x_ref[...], 0.0, 1.0)
    )


def _kernel_tc(x):
    B, C, H, W = x.shape
    mask = jnp.asarray(_line_mask(B, H, W))
    HB = 64
    grid = (B, H // HB)
    return pl.pallas_call(
        _fused_kernel,
        grid=grid,
        in_specs=[
            pl.BlockSpec((1, C, HB, W), lambda b, h: (b, 0, h, 0)),
            pl.BlockSpec((1, 1, HB, W), lambda b, h: (b, 0, h, 0)),
        ],
        out_specs=pl.BlockSpec((1, C, HB, W), lambda b, h: (b, 0, h, 0)),
        out_shape=jax.ShapeDtypeStruct((B, C, H, W), x.dtype),
    )(x, mask)


def kernel(x):
    return _kernel_sc(x)


# speedup vs baseline: 10.2016x; 10.1216x over previous
import functools

import jax
import jax.numpy as jnp
import numpy as np
from jax import lax
from jax.experimental import pallas as pl
from jax.experimental.pallas import tpu as pltpu
from jax.experimental.pallas import tpu_sc as plsc

_NC = 2
_NS = 16
_NW = _NC * _NS
_SUB = 32768


def _line_points(x0, y0, x1, y1, H, W):
    pts = []
    dx, dy = abs(x1 - x0), abs(y1 - y0)
    sx = 1 if x0 < x1 else -1
    sy = 1 if y0 < y1 else -1
    err = dx - dy
    cx, cy = x0, y0
    for _ in range(max(dx, dy) + 1):
        if 0 <= cy < H and 0 <= cx < W:
            pts.append((cy, cx))
        e2 = 2 * err
        if e2 > -dy:
            err -= dy
            cx += sx
        if e2 < dx:
            err += dx
            cy += sy
    return pts


@functools.lru_cache(maxsize=None)
def _line_mask(B, H, W):
    rng = np.random.default_rng(0)
    mask = np.zeros((B, 1, H, W), dtype=np.bool_)
    for b in range(B):
        for _ in range(6):
            y0 = int(rng.integers(0, H))
            x0 = int(rng.integers(0, W))
            y1 = int(rng.integers(0, H))
            x1 = int(rng.integers(0, W))
            for (cy, cx) in _line_points(x0, y0, x1, y1, H, W):
                mask[b, 0, cy, cx] = True
    return mask


def _fused_kernel(x_ref, m_ref, o_ref):
    o_ref[...] = jnp.where(
        m_ref[...], jnp.float32(0.05), jnp.clip(x_ref[...], 0.0, 1.0)
    )


def kernel(x):
    B, C, H, W = x.shape
    mask = jnp.asarray(_line_mask(B, H, W))
    HB = 64
    grid = (B, H // HB)
    return pl.pallas_call(
        _fused_kernel,
        grid=grid,
        in_specs=[
            pl.BlockSpec((1, C, HB, W), lambda b, h: (b, 0, h, 0)),
            pl.BlockSpec((1, 1, HB, W), lambda b, h: (b, 0, h, 0)),
        ],
        out_specs=pl.BlockSpec((1, C, HB, W), lambda b, h: (b, 0, h, 0)),
        out_shape=jax.ShapeDtypeStruct((B, C, H, W), x.dtype),
    )(x, mask)


@functools.lru_cache(maxsize=None)
def _scatter_indices(B, C, H, W):
    mask = _line_mask(B, H, W)[:, 0]
    bs, ys, xs = np.nonzero(mask)
    flat = (bs[:, None] * C + np.arange(C)[None, :]) * (H * W) + (
        ys * W + xs
    )[:, None]
    flat = np.sort(flat.ravel()).astype(np.int64)
    chunk = (B * C * H * W) // _NW
    per_tile = [flat[(flat >= w * chunk) & (flat < (w + 1) * chunk)]
                for w in range(_NW)]
    nrows = max(1, -(-max(len(p) for p in per_tile) // 128))
    out = np.empty((_NW, nrows * 128), dtype=np.int32)
    for w, p in enumerate(per_tile):
        pad_val = p[-1] if len(p) else w * chunk
        out[w, :len(p)] = p
        out[w, len(p):] = pad_val
    return out


def _sc_body(nsub, chunk, x_hbm, idx_hbm, val_hbm, out_hbm,
             buf0, buf1, idx_v, val_v,
             sem_in0a, sem_in0b, sem_in1a, sem_in1b,
             sem_out0a, sem_out0b, sem_out1a, sem_out1b, sem_sc):
    sem_in0 = (sem_in0a, sem_in0b)
    sem_in1 = (sem_in1a, sem_in1b)
    sem_out0 = (sem_out0a, sem_out0b)
    sem_out1 = (sem_out1a, sem_out1b)
    cid = lax.axis_index("c")
    sid = lax.axis_index("s")
    wid = sid * _NC + cid
    base = wid * chunk

    pltpu.sync_copy(idx_hbm.at[wid], idx_v)
    pltpu.sync_copy(val_hbm, val_v)

    half = _SUB // 2

    class _Pair:
        def __init__(self, a, b):
            self._a, self._b = a, b

        def start(self):
            self._a.start()
            self._b.start()

        def wait(self):
            self._a.wait()
            self._b.wait()

    def cp_in(g, buf, sem):
        lo = base + g * _SUB
        return _Pair(
            pltpu.make_async_copy(
                x_hbm.at[pl.ds(lo, half)], buf.at[pl.ds(0, half)], sem[0]),
            pltpu.make_async_copy(
                x_hbm.at[pl.ds(lo + half, half)],
                buf.at[pl.ds(half, half)], sem[1]))

    def cp_out(g, buf, sem):
        lo = base + g * _SUB
        return _Pair(
            pltpu.make_async_copy(
                buf.at[pl.ds(0, half)], out_hbm.at[pl.ds(lo, half)], sem[0]),
            pltpu.make_async_copy(
                buf.at[pl.ds(half, half)],
                out_hbm.at[pl.ds(lo + half, half)], sem[1]))

    def clip_buf(buf):
        @plsc.parallel_loop(0, _SUB // 16, unroll=8)
        def _(i):
            v = buf[pl.ds(i * 16, 16)]
            buf[pl.ds(i * 16, 16)] = jnp.minimum(jnp.maximum(v, 0.0), 1.0)

    cp_in(0, buf0, sem_in0).start()

    @pl.loop(0, nsub, step=2)
    def _(g):
        cp_in(g, buf0, sem_in0).wait()

        @pl.when(g + 1 < nsub)
        def _():
            @pl.when(g > 0)
            def _():
                cp_out(g - 1, buf1, sem_out1).wait()
            cp_in(g + 1, buf1, sem_in1).start()

        clip_buf(buf0)
        cp_out(g, buf0, sem_out0).start()

        @pl.when(g + 1 < nsub)
        def _():
            cp_in(g + 1, buf1, sem_in1).wait()

            @pl.when(g + 2 < nsub)
            def _():
                cp_out(g, buf0, sem_out0).wait()
                cp_in(g + 2, buf0, sem_in0).start()

            clip_buf(buf1)
            cp_out(g + 1, buf1, sem_out1).start()

    cp_out(nsub - 2, buf0, sem_out0).wait()
    cp_out(nsub - 1, buf1, sem_out1).wait()

    pltpu.async_copy(val_v, out_hbm.at[idx_v], sem_sc).wait()


def _kernel_sc(x):
    B, C, H, W = x.shape
    n = B * C * H * W
    chunk = n // _NW
    assert chunk % _SUB == 0
    nsub = chunk // _SUB
    idx = jnp.asarray(_scatter_indices(B, C, H, W))
    nidx = idx.shape[1]
    vals = jnp.full((nidx,), 0.05, dtype=jnp.float32)
    mesh = plsc.VectorSubcoreMesh(
        core_axis_name="c", subcore_axis_name="s",
        num_cores=_NC, num_subcores=_NS)
    run = pl.kernel(
        functools.partial(_sc_body, nsub, chunk),
        out_type=jax.ShapeDtypeStruct((n,), jnp.float32),
        mesh=mesh,
        scratch_types=[
            pltpu.VMEM((_SUB,), jnp.float32),
            pltpu.VMEM((_SUB,), jnp.float32),
            pltpu.VMEM((nidx,), jnp.int32),
            pltpu.VMEM((nidx,), jnp.float32),
        ] + [pltpu.SemaphoreType.DMA] * 9,
    )
    out = run(x.reshape(n), idx, vals)
    return out.reshape(B, C, H, W)
